# baseline (speedup 1.0000x reference)
import jax
import jax.numpy as jnp
from jax import lax
from jax.experimental import pallas as pl
from jax.experimental.pallas import tpu as pltpu
from jax.experimental.pallas import tpu_sc as plsc

BATCH = 16384
LANES = 16
NUM_WORKERS = 16
B_PER_W = BATCH // NUM_WORKERS
CHUNKS = (512, 512)

TWO_LN2 = 2.0 * 0.6931471805599453
BIAS_K = 127.0 * TWO_LN2
E0 = 2.0 * 0.99744885
E1 = 2.0 * -0.47130057
E2 = 2.0 * 0.22568444
E3 = 2.0 * -0.05875647


def _reward(t, r, av):
    bits = lax.bitcast_convert_type(r + 1e-05, jnp.int32)
    ef = (bits >> 23).astype(jnp.float32)
    z = lax.bitcast_convert_type(
        (bits & 0x007FFFFF) | 0x3F800000, jnp.float32) - 1.0
    p = z * (E0 + z * (E1 + z * (E2 + z * E3)))
    move = jnp.where(av != 0, jnp.float32(0.1), jnp.float32(0.0))
    return (100.0 * t + BIAS_K - move) - ef * TWO_LN2 - p


def _sc_body(x_hbm, a_hbm, rho_hbm, theta_hbm, out_hbm,
             idx_v, a_v, tm_v, r_v, sem_a, sem_out, sems_g):
    wid = lax.axis_index("s")
    base = wid * B_PER_W
    pltpu.sync_copy(x_hbm.at[pl.ds(base, B_PER_W)], idx_v)
    gathers = []
    off = 0
    for h, ch in enumerate(CHUNKS):
        sl = pl.ds(off, ch)
        gt = pltpu.async_copy(theta_hbm.at[idx_v.at[sl]], tm_v.at[sl],
                              sems_g.at[h])
        gr = pltpu.async_copy(rho_hbm.at[idx_v.at[sl]], r_v.at[sl],
                              sems_g.at[h])
        gathers.append((gt, gr))
        off += ch
    ca = pltpu.async_copy(a_hbm.at[pl.ds(base, B_PER_W)], a_v, sem_a)
    ca.wait()
    out_copies = []
    off = 0
    for h, ch in enumerate(CHUNKS):
        gt, gr = gathers[h]
        gt.wait()
        gr.wait()
        for i in range(ch // LANES):
            sl = pl.ds(off + i * LANES, LANES)
            r_v[sl] = _reward(tm_v[sl], r_v[sl], a_v[sl])
        out_copies.append(
            pltpu.async_copy(r_v.at[pl.ds(off, ch)],
                             out_hbm.at[pl.ds(base + off, ch)], sem_out))
        off += ch
    for c in out_copies:
        c.wait()


@jax.jit
def kernel(x, a, rho, theta_map):
    mesh = plsc.VectorSubcoreMesh(core_axis_name="c", subcore_axis_name="s",
                                  num_cores=1)
    run = pl.kernel(
        _sc_body,
        mesh=mesh,
        out_type=jax.ShapeDtypeStruct((BATCH,), jnp.float32),
        scratch_types=[
            pltpu.VMEM((B_PER_W,), jnp.int32),
            pltpu.VMEM((B_PER_W,), jnp.int32),
            pltpu.VMEM((B_PER_W,), jnp.float32),
            pltpu.VMEM((B_PER_W,), jnp.float32),
            pltpu.SemaphoreType.DMA,
            pltpu.SemaphoreType.DMA,
            pltpu.SemaphoreType.DMA((2,)),
        ],
    )
    return run(x, a, rho, theta_map)


# speedup vs baseline: 1.0038x; 1.0038x over previous
import jax
import jax.numpy as jnp
from jax import lax
from jax.experimental import pallas as pl
from jax.experimental.pallas import tpu as pltpu
from jax.experimental.pallas import tpu_sc as plsc

BATCH = 16384
LANES = 16
NUM_WORKERS = 32
B_PER_W = BATCH // NUM_WORKERS
CHUNKS = (256, 256)

TWO_LN2 = 2.0 * 0.6931471805599453
BIAS_K = 127.0 * TWO_LN2
E0 = 2.0 * 0.99744885
E1 = 2.0 * -0.47130057
E2 = 2.0 * 0.22568444
E3 = 2.0 * -0.05875647


def _reward(t, r, av):
    bits = lax.bitcast_convert_type(r + 1e-05, jnp.int32)
    ef = (bits >> 23).astype(jnp.float32)
    z = lax.bitcast_convert_type(
        (bits & 0x007FFFFF) | 0x3F800000, jnp.float32) - 1.0
    p = z * (E0 + z * (E1 + z * (E2 + z * E3)))
    move = jnp.where(av != 0, jnp.float32(0.1), jnp.float32(0.0))
    return (100.0 * t + BIAS_K - move) - ef * TWO_LN2 - p


def _sc_body(x_hbm, a_hbm, rho_hbm, theta_hbm, out_hbm,
             idx_v, a_v, tm_v, r_v, sem_a, sem_out, sems_g):
    wid = lax.axis_index("s") * 2 + lax.axis_index("c")
    base = wid * B_PER_W
    pltpu.sync_copy(x_hbm.at[pl.ds(base, B_PER_W)], idx_v)
    gathers = []
    off = 0
    for h, ch in enumerate(CHUNKS):
        sl = pl.ds(off, ch)
        gt = pltpu.async_copy(theta_hbm.at[idx_v.at[sl]], tm_v.at[sl],
                              sems_g.at[h])
        gr = pltpu.async_copy(rho_hbm.at[idx_v.at[sl]], r_v.at[sl],
                              sems_g.at[h])
        gathers.append((gt, gr))
        off += ch
    ca = pltpu.async_copy(a_hbm.at[pl.ds(base, B_PER_W)], a_v, sem_a)
    ca.wait()
    out_copies = []
    off = 0
    for h, ch in enumerate(CHUNKS):
        gt, gr = gathers[h]
        gt.wait()
        gr.wait()
        for i in range(ch // LANES):
            sl = pl.ds(off + i * LANES, LANES)
            r_v[sl] = _reward(tm_v[sl], r_v[sl], a_v[sl])
        out_copies.append(
            pltpu.async_copy(r_v.at[pl.ds(off, ch)],
                             out_hbm.at[pl.ds(base + off, ch)], sem_out))
        off += ch
    for c in out_copies:
        c.wait()


@jax.jit
def kernel(x, a, rho, theta_map):
    mesh = plsc.VectorSubcoreMesh(core_axis_name="c", subcore_axis_name="s")
    run = pl.kernel(
        _sc_body,
        mesh=mesh,
        out_type=jax.ShapeDtypeStruct((BATCH,), jnp.float32),
        scratch_types=[
            pltpu.VMEM((B_PER_W,), jnp.int32),
            pltpu.VMEM((B_PER_W,), jnp.int32),
            pltpu.VMEM((B_PER_W,), jnp.float32),
            pltpu.VMEM((B_PER_W,), jnp.float32),
            pltpu.SemaphoreType.DMA,
            pltpu.SemaphoreType.DMA,
            pltpu.SemaphoreType.DMA((2,)),
        ],
    )
    return run(x, a, rho, theta_map)
